# baseline (speedup 1.0000x reference)
import jax
import jax.numpy as jnp
from jax.experimental import pallas as pl
from jax.experimental.pallas import tpu as pltpu

_N, _M, _D = 10000, 64, 32
_BN = 1000
_NB = _N // _BN

_F32 = jnp.float32
_CONTRACT0 = (((0,), (0,)), ((), ()))
_CONTRACT_01 = (((0,), (1,)), ((), ()))
_CONTRACT_11 = (((1,), (1,)), ((), ()))


def _fused_kernel(x0_ref, b_ref, w_ref, out_ref, t_ref, browc_ref):
    j = pl.program_id(0)

    @pl.when(j == 0)
    def _reduce():
        B = b_ref[...]
        x0 = x0_ref[...]
        w = w_ref[...]

        x1_1 = jax.lax.dot_general(B, x0, _CONTRACT0,
                                   preferred_element_type=_F32)
        G = jax.lax.dot_general(B, B, _CONTRACT0,
                                preferred_element_type=_F32)
        x1_2 = jnp.dot(G, jnp.dot(x1_1, w[0:32],
                                  preferred_element_type=_F32),
                       preferred_element_type=_F32)
        y = jnp.dot(x1_2, w[32:64], preferred_element_type=_F32)

        w3r = w[160:161]
        u = jax.lax.dot_general(w[128:160], w3r, _CONTRACT_11,
                                preferred_element_type=_F32)
        av = jnp.dot(w[64:96], u, preferred_element_type=_F32)
        bv = jnp.dot(w[96:128], u, preferred_element_type=_F32)
        c = (jnp.dot(w[161:162], u, preferred_element_type=_F32)[0, 0]
             + jax.lax.dot_general(w[162:163], w3r, _CONTRACT_11,
                                   preferred_element_type=_F32)[0, 0]
             + w[163, 0])

        t_ref[...] = jnp.dot(y, av, preferred_element_type=_F32)
        browc_ref[...] = jax.lax.dot_general(
            bv, x1_2, _CONTRACT_01, preferred_element_type=_F32) + c

    @pl.when(j > 0)
    def _emit():
        base = (j - 1) * _BN
        B = b_ref[pl.ds(base, _BN), :]
        a_col = jnp.dot(B, t_ref[...], preferred_element_type=_F32)
        out_ref[...] = jnp.where(B != 0, a_col + browc_ref[...], 0.0)


def kernel(x_0, incidence_1, W1, W2, Wm1, bm1, Wm2, bm2, Wm3, bm3):
    n, m = incidence_1.shape
    d = x_0.shape[1]
    wpack = jnp.concatenate([
        W1, W2, Wm1, Wm2,
        Wm3.T,
        bm1[None, :], bm2[None, :],
        jnp.pad(bm3[None, :], ((0, 0), (0, d - bm3.shape[0]))),
    ], axis=0)
    full = lambda a: pl.BlockSpec(a.shape, lambda j: (0,) * a.ndim)
    return pl.pallas_call(
        _fused_kernel,
        grid=(1 + _NB,),
        in_specs=[
            full(x_0),
            full(incidence_1),
            full(wpack),
        ],
        out_specs=pl.BlockSpec((_BN, m),
                               lambda j: (jnp.maximum(j - 1, 0), 0)),
        out_shape=jax.ShapeDtypeStruct((n, m), jnp.float32),
        scratch_shapes=[
            pltpu.VMEM((m, 1), jnp.float32),
            pltpu.VMEM((1, m), jnp.float32),
        ],
    )(x_0, incidence_1, wpack)


# speedup vs baseline: 1.0683x; 1.0683x over previous
import jax
import jax.numpy as jnp
from jax.experimental import pallas as pl

_F32 = jnp.float32
_CONTRACT0 = (((0,), (0,)), ((), ()))
_CONTRACT_01 = (((0,), (1,)), ((), ()))
_CONTRACT_11 = (((1,), (1,)), ((), ()))


def _fused_kernel(x0_ref, b_ref, w_ref, out_ref):
    B = b_ref[...]
    x0 = x0_ref[...]
    w = w_ref[...]

    x1_1 = jax.lax.dot_general(B, x0, _CONTRACT0,
                               preferred_element_type=_F32)
    G = jax.lax.dot_general(B, B, _CONTRACT0,
                            preferred_element_type=_F32)
    x1_2 = jnp.dot(G, jnp.dot(x1_1, w[0:32],
                              preferred_element_type=_F32),
                   preferred_element_type=_F32)
    y = jnp.dot(x1_2, w[32:64], preferred_element_type=_F32)

    w3r = w[160:161]
    u = jax.lax.dot_general(w[128:160], w3r, _CONTRACT_11,
                            preferred_element_type=_F32)
    av = jnp.dot(w[64:96], u, preferred_element_type=_F32)
    bv = jnp.dot(w[96:128], u, preferred_element_type=_F32)
    c = (jnp.dot(w[161:162], u, preferred_element_type=_F32)[0, 0]
         + jax.lax.dot_general(w[162:163], w3r, _CONTRACT_11,
                               preferred_element_type=_F32)[0, 0]
         + w[163, 0])

    t = jnp.dot(y, av, preferred_element_type=_F32)
    browc = jax.lax.dot_general(bv, x1_2, _CONTRACT_01,
                                preferred_element_type=_F32) + c

    a_col = jnp.dot(B, t, preferred_element_type=_F32)
    out_ref[...] = jnp.where(B != 0, a_col + browc, 0.0)


def kernel(x_0, incidence_1, W1, W2, Wm1, bm1, Wm2, bm2, Wm3, bm3):
    n, m = incidence_1.shape
    d = x_0.shape[1]
    wpack = jnp.concatenate([
        W1, W2, Wm1, Wm2,
        Wm3.T,
        bm1[None, :], bm2[None, :],
        jnp.pad(bm3[None, :], ((0, 0), (0, d - bm3.shape[0]))),
    ], axis=0)
    return pl.pallas_call(
        _fused_kernel,
        out_shape=jax.ShapeDtypeStruct((n, m), jnp.float32),
    )(x_0, incidence_1, wpack)
